# Initial kernel scaffold; baseline (speedup 1.0000x reference)
import jax
import jax.numpy as jnp
from jax.experimental import pallas as pl


def kernel(pattern_ids, chord_ids, bar_numbers, beat_numbers, pattern_table, chord_table, bar_table, beat_table, ln_gamma, ln_beta):
    raise NotImplementedError("write your pallas kernel here")


# speedup vs baseline: 3.8303x; 3.8303x over previous
import functools

import jax
import jax.numpy as jnp
from jax import lax
from jax.experimental import pallas as pl
from jax.experimental.pallas import tpu as pltpu
from jax.experimental.pallas import tpu_sc as plsc

HIDDEN = 128
N_TOK = 4096 * 200
NW = 32
PER_W = N_TOK // NW
CH = 128
N_CH = PER_W // CH
EPS = 1e-12


_GDN = lax.GatherDimensionNumbers(
    offset_dims=(), collapsed_slice_dims=(0,), start_index_map=(0,))


def _lane_allreduce_sum(x):
    ids = lax.iota(jnp.int32, 16)
    for k in (1, 2, 4, 8):
        perm = lax.bitwise_xor(ids, k)
        x = x + lax.gather(x, perm[:, None], _GDN, slice_sizes=(1,),
                           mode=lax.GatherScatterMode.PROMISE_IN_BOUNDS)
    return x


def _rsqrt(x):
    i = lax.bitcast_convert_type(x, jnp.int32)
    i = 0x5F3759DF - lax.shift_right_logical(i, 1)
    y = lax.bitcast_convert_type(i, jnp.float32)
    for _ in range(3):
        y = y * (1.5 - 0.5 * x * y * y)
    return y


def _sc_embed_ln(pid_h, cid_h, brid_h, btid_h,
                 pat_t, cho_t, bar_t, bea_t,
                 gam_h, bet_h, out_h,
                 pid_v, cid_v, brid_v, btid_v,
                 pat_v, cho_v, bar_v, bea_v, out_v,
                 gam_v, bet_v, sem):
    wid = lax.axis_index("s") * 2 + lax.axis_index("c")

    pltpu.sync_copy(gam_h, gam_v)
    pltpu.sync_copy(bet_h, bet_v)
    gamma = [gam_v[pl.ds(j * 16, 16)] for j in range(8)]
    beta = [bet_v[pl.ds(j * 16, 16)] for j in range(8)]

    def chunk(g, _):
        base = wid * PER_W + g * CH
        pltpu.sync_copy(pid_h.at[pl.ds(base, CH)], pid_v)
        pltpu.sync_copy(cid_h.at[pl.ds(base, CH)], cid_v)
        pltpu.sync_copy(brid_h.at[pl.ds(base, CH)], brid_v)
        pltpu.sync_copy(btid_h.at[pl.ds(base, CH)], btid_v)

        c1 = pltpu.async_copy(pat_t.at[pid_v], pat_v, sem)
        c2 = pltpu.async_copy(cho_t.at[cid_v], cho_v, sem)
        c3 = pltpu.async_copy(bar_t.at[brid_v], bar_v, sem)
        c4 = pltpu.async_copy(bea_t.at[btid_v], bea_v, sem)
        c1.wait()
        c2.wait()
        c3.wait()
        c4.wait()

        def tok(t, carry):
            xs = []
            for j in range(8):
                sl = pl.ds(j * 16, 16)
                x = pat_v[t, sl] + cho_v[t, sl] + bar_v[t, sl] + bea_v[t, sl]
                xs.append(x)
            s = xs[0]
            for j in range(1, 8):
                s = s + xs[j]
            sq = xs[0] * xs[0]
            for j in range(1, 8):
                sq = sq + xs[j] * xs[j]
            mean = _lane_allreduce_sum(s) * (1.0 / HIDDEN)
            ex2 = _lane_allreduce_sum(sq) * (1.0 / HIDDEN)
            inv = _rsqrt(ex2 - mean * mean + EPS)
            for j in range(8):
                out_v[t, pl.ds(j * 16, 16)] = (
                    (xs[j] - mean) * inv * gamma[j] + beta[j])
            return carry

        lax.fori_loop(0, CH, tok, 0, unroll=False)
        pltpu.sync_copy(out_v, out_h.at[pl.ds(base, CH)])
        return _

    lax.fori_loop(0, N_CH, chunk, 0, unroll=False)


@jax.jit
def _run(pid, cid, brid, btid, pat_t, cho_t, bar_t, bea_t, gam, bet):
    mesh = plsc.VectorSubcoreMesh(core_axis_name="c", subcore_axis_name="s")
    f = functools.partial(
        pl.kernel,
        out_type=jax.ShapeDtypeStruct((N_TOK, HIDDEN), jnp.float32),
        mesh=mesh,
        scratch_types=[
            pltpu.VMEM((CH,), jnp.int32),
            pltpu.VMEM((CH,), jnp.int32),
            pltpu.VMEM((CH,), jnp.int32),
            pltpu.VMEM((CH,), jnp.int32),
            pltpu.VMEM((CH, HIDDEN), jnp.float32),
            pltpu.VMEM((CH, HIDDEN), jnp.float32),
            pltpu.VMEM((CH, HIDDEN), jnp.float32),
            pltpu.VMEM((CH, HIDDEN), jnp.float32),
            pltpu.VMEM((CH, HIDDEN), jnp.float32),
            pltpu.VMEM((HIDDEN,), jnp.float32),
            pltpu.VMEM((HIDDEN,), jnp.float32),
            pltpu.SemaphoreType.DMA,
        ],
    )(_sc_embed_ln)
    return f(pid, cid, brid, btid, pat_t, cho_t, bar_t, bea_t, gam, bet)


def kernel(pattern_ids, chord_ids, bar_numbers, beat_numbers,
           pattern_table, chord_table, bar_table, beat_table,
           ln_gamma, ln_beta):
    shp = pattern_ids.shape
    pid = pattern_ids.reshape(-1).astype(jnp.int32)
    cid = chord_ids.reshape(-1).astype(jnp.int32)
    brid = bar_numbers.reshape(-1).astype(jnp.int32)
    btid = beat_numbers.reshape(-1).astype(jnp.int32)
    out = _run(pid, cid, brid, btid,
               pattern_table, chord_table, bar_table, beat_table,
               ln_gamma, ln_beta)
    return out.reshape(shp + (HIDDEN,))
